# baseline (speedup 1.0000x reference)
import functools

import jax
import jax.numpy as jnp
from jax import lax
from jax.experimental import pallas as pl
from jax.experimental.pallas import tpu as pltpu
from jax.experimental.pallas import tpu_sc as plsc

_BOOST = 3.0
_K1 = 200
_NBI = 64
_CB = 8
_L = 16


@functools.lru_cache(maxsize=None)
def _build_repack(V, D, nc, ns):
    NW = nc * ns
    assert V % _K1 == 0 and _K1 % 8 == 0
    total = V // _K1
    pairs = (total // NW + 2) // 2
    mesh = plsc.VectorSubcoreMesh(
        core_axis_name="c", subcore_axis_name="s", num_cores=nc, num_subcores=ns
    )

    @functools.partial(
        pl.kernel,
        out_type=jax.ShapeDtypeStruct((V, 2 * D), jnp.float32),
        mesh=mesh,
        scratch_types=[
            pltpu.VMEM((_K1, D), jnp.float32),
            pltpu.VMEM((_K1, D), jnp.float32),
            pltpu.VMEM((_K1, 2 * D), jnp.float32),
            pltpu.SemaphoreType.DMA,
            pltpu.SemaphoreType.DMA,
        ],
    )
    def k(w_hbm, scr_hbm, t0, t1, wide, s0, s1):
        wid = lax.axis_index("s") * nc + lax.axis_index("c")
        bufs = (t0, t1)
        sems = (s0, s1)

        def read_start(c, slot):
            pltpu.async_copy(
                w_hbm.at[pl.ds(c * _K1, _K1)], bufs[slot], sems[slot]
            )

        def read_wait(slot):
            pltpu.make_async_copy(
                w_hbm.at[pl.ds(0, _K1)], bufs[slot], sems[slot]
            ).wait()

        def process(c, slot):
            buf = bufs[slot]

            @plsc.parallel_loop(0, _K1, 1, unroll=4)
            def _(r):
                for cc in range(D // _L):
                    wide[r, pl.ds(cc * _L, _L)] = buf[r, pl.ds(cc * _L, _L)]

            pltpu.sync_copy(wide, scr_hbm.at[pl.ds(c * _K1, _K1)])

        @pl.when(wid < total)
        def _():
            read_start(wid, 0)

        @pl.loop(0, pairs)
        def _(t):
            c0 = wid + NW * (2 * t)
            c1 = c0 + NW
            c2 = c1 + NW

            @pl.when(c0 < total)
            def _():
                read_wait(0)

                @pl.when(c1 < total)
                def _():
                    read_start(c1, 1)

                process(c0, 0)

            @pl.when(c1 < total)
            def _():
                read_wait(1)

                @pl.when(c2 < total)
                def _():
                    read_start(c2, 0)

                process(c1, 1)

    return k


@functools.lru_cache(maxsize=None)
def _build_gather(NB, S, V, D, nc, ns):
    NW = nc * ns
    assert NB % NW == 0
    nb = NB // NW
    assert nb % _NBI == 0 and _NBI % (2 * _CB) == 0
    mesh = plsc.VectorSubcoreMesh(
        core_axis_name="c", subcore_axis_name="s", num_cores=nc, num_subcores=ns
    )

    @functools.partial(
        pl.kernel,
        out_type=jax.ShapeDtypeStruct((NB, S, D), jnp.float32),
        mesh=mesh,
        scratch_types=[
            pltpu.VMEM((_NBI, S), jnp.int32),
            pltpu.VMEM((_CB * S, 2 * D), jnp.float32),
            pltpu.VMEM((_CB * S, 2 * D), jnp.float32),
            pltpu.VMEM((_CB, S, D), jnp.float32),
            pltpu.SemaphoreType.DMA,
            pltpu.SemaphoreType.DMA,
        ],
    )
    def k(x_hbm, scr_hbm, out_hbm, idx_v, fb0, fb1, ost, s0, s1):
        wid = lax.axis_index("s") * nc + lax.axis_index("c")
        base = wid * nb
        fbs = (fb0, fb1)
        sems = (s0, s1)

        def gather_start(cc, slot):
            for j in range(_CB):
                pltpu.async_copy(
                    scr_hbm.at[idx_v.at[cc * _CB + j, :]],
                    fbs[slot].at[pl.ds(j * S, S)],
                    sems[slot],
                )

        def gather_wait(slot):
            for j in range(_CB):
                pltpu.make_async_copy(
                    scr_hbm.at[idx_v.at[0, :]],
                    fbs[slot].at[pl.ds(j * S, S)],
                    sems[slot],
                ).wait()

        def compact_scale(slot):
            fb = fbs[slot]

            @plsc.parallel_loop(0, _CB * S, 1, unroll=2)
            def _(r):
                j = r // S
                s = r % S
                for c in range(D // _L):
                    ost[j, s, pl.ds(c * _L, _L)] = (
                        fb[r, pl.ds(c * _L, _L)] * _BOOST
                    )

        def flush(b0):
            pltpu.sync_copy(ost, out_hbm.at[pl.ds(b0, _CB)])

        @pl.loop(0, nb // _NBI)
        def _(ib):
            blk = base + ib * _NBI
            pltpu.sync_copy(x_hbm.at[pl.ds(blk, _NBI)], idx_v)
            gather_start(0, 0)

            @pl.loop(0, _NBI // _CB, step=2)
            def _(h):
                gather_start(h + 1, 1)
                gather_wait(0)
                compact_scale(0)
                flush(blk + h * _CB)

                @pl.when(h + 2 < _NBI // _CB)
                def _():
                    gather_start(h + 2, 0)

                gather_wait(1)
                compact_scale(1)
                flush(blk + (h + 1) * _CB)

    return k


def kernel(x, weight):
    V, D = weight.shape
    NB, S = x.shape
    info = plsc.get_sparse_core_info()
    nc, ns = info.num_cores, info.num_subcores
    scr = _build_repack(V, D, nc, ns)(weight)
    return _build_gather(NB, S, V, D, nc, ns)(x, scr)


# speedup vs baseline: 1.1790x; 1.1790x over previous
import functools

import jax
import jax.numpy as jnp
from jax import lax
from jax.experimental import pallas as pl
from jax.experimental.pallas import tpu as pltpu
from jax.experimental.pallas import tpu_sc as plsc

_BOOST = 3.0
_NBI = 64
_CB = 8
_L = 16


@functools.lru_cache(maxsize=None)
def _build_gather(NB, S, V, D, nc, ns):
    NW = nc * ns
    assert NB % NW == 0
    nb = NB // NW
    assert nb % _NBI == 0 and _NBI % (2 * _CB) == 0
    mesh = plsc.VectorSubcoreMesh(
        core_axis_name="c", subcore_axis_name="s", num_cores=nc, num_subcores=ns
    )

    @functools.partial(
        pl.kernel,
        out_type=jax.ShapeDtypeStruct((NB, S, D), jnp.float32),
        mesh=mesh,
        scratch_types=[
            pltpu.VMEM((_NBI, S), jnp.int32),
            pltpu.VMEM((_CB * S, 2 * D), jnp.float32),
            pltpu.VMEM((_CB * S, 2 * D), jnp.float32),
            pltpu.VMEM((_CB, S, D), jnp.float32),
            pltpu.SemaphoreType.DMA,
            pltpu.SemaphoreType.DMA,
        ],
    )
    def k(x_hbm, w_hbm, out_hbm, idx_v, fb0, fb1, ost, s0, s1):
        wid = lax.axis_index("s") * nc + lax.axis_index("c")
        base = wid * nb
        fbs = (fb0, fb1)
        sems = (s0, s1)

        def gather_start(cc, slot):
            for j in range(_CB):
                pltpu.async_copy(
                    w_hbm.at[idx_v.at[cc * _CB + j, :]],
                    fbs[slot].at[pl.ds(j * S, S)],
                    sems[slot],
                )

        def gather_wait(slot):
            for j in range(_CB):
                pltpu.make_async_copy(
                    w_hbm.at[idx_v.at[0, :]],
                    fbs[slot].at[pl.ds(j * S, S)],
                    sems[slot],
                ).wait()

        def compact_scale(slot):
            fb = fbs[slot]

            @plsc.parallel_loop(0, _CB * S, 1, unroll=2)
            def _(r):
                j = r // S
                s = r % S
                for c in range(D // _L):
                    ost[j, s, pl.ds(c * _L, _L)] = (
                        fb[r, pl.ds(c * _L, _L)] * _BOOST
                    )

        def flush(b0):
            pltpu.sync_copy(ost, out_hbm.at[pl.ds(b0, _CB)])

        @pl.loop(0, nb // _NBI)
        def _(ib):
            blk = base + ib * _NBI
            pltpu.sync_copy(x_hbm.at[pl.ds(blk, _NBI)], idx_v)
            gather_start(0, 0)

            @pl.loop(0, _NBI // _CB, step=2)
            def _(h):
                gather_start(h + 1, 1)
                gather_wait(0)
                compact_scale(0)
                flush(blk + h * _CB)

                @pl.when(h + 2 < _NBI // _CB)
                def _():
                    gather_start(h + 2, 0)

                gather_wait(1)
                compact_scale(1)
                flush(blk + (h + 1) * _CB)

    return k


def kernel(x, weight):
    V, D = weight.shape
    NB, S = x.shape
    wide = jnp.pad(weight, ((0, 0), (0, D)))
    info = plsc.get_sparse_core_info()
    nc, ns = info.num_cores, info.num_subcores
    return _build_gather(NB, S, V, D, nc, ns)(x, wide)


# speedup vs baseline: 1.4531x; 1.2325x over previous
import functools

import jax
import jax.numpy as jnp
from jax import lax
from jax.experimental import pallas as pl
from jax.experimental.pallas import tpu as pltpu
from jax.experimental.pallas import tpu_sc as plsc

_BOOST = 3.0
_NBI = 64
_CB = 8
_L = 16


@functools.lru_cache(maxsize=None)
def _build_gather(NB, S, V, D, nc, ns):
    NW = nc * ns
    assert NB % NW == 0
    nb = NB // NW
    assert nb % _NBI == 0 and _NBI % (2 * _CB) == 0
    mesh = plsc.VectorSubcoreMesh(
        core_axis_name="c", subcore_axis_name="s", num_cores=nc, num_subcores=ns
    )

    @functools.partial(
        pl.kernel,
        out_type=jax.ShapeDtypeStruct((NB, S, D), jnp.float32),
        mesh=mesh,
        scratch_types=[
            pltpu.SMEM((_NBI * S,), jnp.int32),
            pltpu.VMEM((_NBI * S,), jnp.int32),
            pltpu.VMEM_SHARED((ns * _NBI * S,), jnp.int32),
            pltpu.VMEM((_CB * S, D), jnp.float32),
            pltpu.VMEM((_CB * S, D), jnp.float32),
            pltpu.VMEM((_CB, S, D), jnp.float32),
            pltpu.SemaphoreType.DMA,
            pltpu.SemaphoreType.DMA,
        ],
    )
    def k(x_hbm, w_hbm, out_hbm, idx_s, idx_v, idx_sh, fb0, fb1, ost, s0, s1):
        sid = lax.axis_index("s")
        wid = sid * nc + lax.axis_index("c")
        base = wid * nb
        fbs = (fb0, fb1)
        sems = (s0, s1)

        def gather_start(cc, slot):
            @pl.loop(0, _CB * S)
            def _(r):
                idx = idx_s[cc * _CB * S + r]
                pltpu.async_copy(
                    w_hbm.at[pl.ds(idx, 1)],
                    fbs[slot].at[pl.ds(r, 1)],
                    sems[slot],
                )

        def gather_wait(slot):
            pltpu.make_async_copy(
                w_hbm.at[pl.ds(0, _CB * S)], fbs[slot], sems[slot]
            ).wait()

        def compact_scale(slot):
            fb = fbs[slot]

            @plsc.parallel_loop(0, _CB * S, 1, unroll=2)
            def _(r):
                j = r // S
                s = r % S
                for c in range(D // _L):
                    ost[j, s, pl.ds(c * _L, _L)] = (
                        fb[r, pl.ds(c * _L, _L)] * _BOOST
                    )

        def flush(b0):
            pltpu.sync_copy(ost, out_hbm.at[pl.ds(b0, _CB)])

        @pl.loop(0, nb // _NBI)
        def _(ib):
            blk = base + ib * _NBI
            sh = idx_sh.at[pl.ds(sid * _NBI * S, _NBI * S)]
            pltpu.sync_copy(x_hbm.at[pl.ds(blk * S, _NBI * S)], idx_v)
            pltpu.sync_copy(idx_v, sh)
            pltpu.sync_copy(sh, idx_s)
            pltpu.sync_copy(sh, idx_s)
            gather_start(0, 0)

            @pl.loop(0, _NBI // _CB, step=2)
            def _(h):
                gather_start(h + 1, 1)
                gather_wait(0)
                compact_scale(0)
                flush(blk + h * _CB)

                @pl.when(h + 2 < _NBI // _CB)
                def _():
                    gather_start(h + 2, 0)

                gather_wait(1)
                compact_scale(1)
                flush(blk + (h + 1) * _CB)

    return k


def kernel(x, weight):
    V, D = weight.shape
    NB, S = x.shape
    info = plsc.get_sparse_core_info()
    nc, ns = info.num_cores, info.num_subcores
    return _build_gather(NB, S, V, D, nc, ns)(x.reshape(-1), weight)
